# baseline (speedup 1.0000x reference)
import functools

import jax
import jax.numpy as jnp
from jax import lax
from jax.experimental import pallas as pl
from jax.experimental.pallas import tpu as pltpu
from jax.experimental.pallas import tpu_sc as plsc

WE = 64
L_CTX = 50
ER_THR = 0.5
EL_THR = 0.5

NC = 2
NS = 16
LANES = 16
CH = 1568


def _make_sc_proj(v):
    mesh = plsc.VectorSubcoreMesh(core_axis_name="c", subcore_axis_name="s",
                                  num_cores=NC, num_subcores=NS)
    span = -(-v // NS)
    span = (span + 15) & ~15
    n_ch = -(-span // CH)
    groups1 = CH // LANES

    @functools.partial(
        pl.kernel,
        out_type=[jax.ShapeDtypeStruct((v,), jnp.float32),
                  jax.ShapeDtypeStruct((v,), jnp.float32)],
        mesh=mesh,
        compiler_params=pltpu.CompilerParams(needs_layout_passes=False),
        scratch_types=[
            pltpu.VMEM((CH * WE,), jnp.float32),
            pltpu.VMEM((CH,), jnp.float32),
            pltpu.VMEM((WE * LANES,), jnp.float32),
        ],
    )
    def sc_proj(wt_hbm, et_hbm, erwb_hbm, elwb_hbm,
                wproj_hbm, eproj_hbm, chunk_v, p_v, wb_v):
        c = lax.axis_index("c")
        s = lax.axis_index("s")
        iota16 = lax.iota(jnp.int32, 16)

        def project(tbl_hbm, wb_hbm, proj_hbm):
            pltpu.sync_copy(wb_hbm, wb_v)
            for k in range(n_ch):
                row0 = jnp.minimum(s * span + k * CH, v - CH)
                pltpu.sync_copy(tbl_hbm.at[pl.ds(row0 * WE, CH * WE)],
                                chunk_v)
                for g in range(groups1):
                    p_v[pl.ds(g * LANES, LANES)] = jnp.zeros(
                        (LANES,), jnp.float32)

                def col_step(cc, carry):
                    wbc = wb_v[pl.ds(cc * LANES, LANES)]
                    base_iv = iota16 * WE + cc

                    def row_step(g, carry2):
                        flat_iv = base_iv + g * (LANES * WE)
                        vals = plsc.load_gather(chunk_v, [flat_iv])
                        plsc.addupdate(p_v.at[pl.ds(g * LANES, LANES)],
                                       vals * wbc)
                        return carry2
                    lax.fori_loop(0, groups1, row_step, 0, unroll=7)
                    return carry
                lax.fori_loop(0, WE, col_step, 0)
                pltpu.sync_copy(p_v, proj_hbm.at[pl.ds(row0, CH)])

        @pl.when(c == 0)
        def _():
            project(wt_hbm, erwb_hbm, wproj_hbm)

        @pl.when(c != 0)
        def _():
            project(et_hbm, elwb_hbm, eproj_hbm)

    return sc_proj


def _make_sc_gather(v, rows, cw):
    mesh = plsc.VectorSubcoreMesh(core_axis_name="c", subcore_axis_name="s",
                                  num_cores=NC, num_subcores=NS)
    groups2 = cw // LANES

    @functools.partial(
        pl.kernel,
        out_type=[jax.ShapeDtypeStruct((rows,), jnp.float32),
                  jax.ShapeDtypeStruct((rows,), jnp.float32)],
        mesh=mesh,
        compiler_params=pltpu.CompilerParams(needs_layout_passes=False),
        scratch_types=[
            pltpu.VMEM((v,), jnp.float32),
            pltpu.VMEM((L_CTX, cw), jnp.int32),
            pltpu.VMEM((cw,), jnp.float32),
        ],
    )
    def sc_gather(wproj_hbm, eproj_hbm, widx_hbm, eidx_hbm,
                  wsum_hbm, esum_hbm, proj_v, idx_v, out_v):
        c = lax.axis_index("c")
        s = lax.axis_index("s")
        base = s * cw

        @pl.when(c == 0)
        def _():
            pltpu.sync_copy(wproj_hbm, proj_v)
            pltpu.sync_copy(widx_hbm.at[:, pl.ds(base, cw)], idx_v)

        @pl.when(c != 0)
        def _():
            pltpu.sync_copy(eproj_hbm, proj_v)
            pltpu.sync_copy(eidx_hbm.at[:, pl.ds(base, cw)], idx_v)

        def row_group(g, carry):
            def ctx_step(l, acc):
                iv = idx_v[l, pl.ds(g * LANES, LANES)]
                return acc + plsc.load_gather(proj_v, [iv])
            acc = lax.fori_loop(0, L_CTX, ctx_step,
                                jnp.zeros((LANES,), jnp.float32),
                                unroll=10)
            out_v[pl.ds(g * LANES, LANES)] = acc
            return carry
        lax.fori_loop(0, groups2, row_group, 0)

        @pl.when(c == 0)
        def _():
            pltpu.sync_copy(out_v, wsum_hbm.at[pl.ds(base, cw)])

        @pl.when(c != 0)
        def _():
            pltpu.sync_copy(out_v, esum_hbm.at[pl.ds(base, cw)])

    return sc_gather


def _epilogue_body(w_ref, e_ref, erb_ref, elb_ref, ecw_ref, cw_ref, cb_ref,
                   o_ref):
    inv = jnp.float32(1.0 / L_CTX)
    er_raw = w_ref[...] * inv + erb_ref[0]
    el_raw = e_ref[...] * inv + elb_ref[0]
    er_s = jnp.maximum(er_raw - ER_THR, 0.0) + ER_THR
    el_s = jnp.maximum(el_raw - EL_THR, 0.0) + ER_THR
    ec = jax.nn.sigmoid(er_s * ecw_ref[0, 0] + el_s * ecw_ref[1, 0])
    o_ref[...] = (er_s * cw_ref[0, 0] + el_s * cw_ref[1, 0]
                  + ec * cw_ref[2, 0] + cb_ref[0])


def _epilogue(wsum, esum, er_b, el_b, ec_W, cluster_W, cluster_b):
    rows = wsum.shape[0]
    w2 = wsum.reshape(rows // 128, 128)
    e2 = esum.reshape(rows // 128, 128)
    smem = pl.BlockSpec(memory_space=pltpu.SMEM)
    out = pl.pallas_call(
        _epilogue_body,
        in_specs=[pl.BlockSpec(w2.shape, lambda: (0, 0)),
                  pl.BlockSpec(e2.shape, lambda: (0, 0)),
                  smem, smem, smem, smem, smem],
        out_specs=pl.BlockSpec(w2.shape, lambda: (0, 0)),
        out_shape=jax.ShapeDtypeStruct(w2.shape, jnp.float32),
    )(w2, e2, er_b, el_b, ec_W, cluster_W, cluster_b)
    return out.reshape(rows, 1)


def kernel(lctx_words, rctx_words, lctx_entities, rctx_entities,
           word_table, entity_table, er_W, er_b, el_W, el_b,
           ec_W, cluster_W, cluster_b):
    b = lctx_words.shape[0]
    rows = 2 * b
    v = word_table.shape[0]

    widx = jnp.concatenate([rctx_words, lctx_words], axis=0)
    eidx = jnp.concatenate([rctx_entities, lctx_entities], axis=0)
    widx_t = widx.T.astype(jnp.int32)
    eidx_t = eidx.T.astype(jnp.int32)

    erw_b = jnp.broadcast_to(er_W.astype(jnp.float32), (WE, LANES)).reshape(-1)
    elw_b = jnp.broadcast_to(el_W.astype(jnp.float32), (WE, LANES)).reshape(-1)

    cw = rows // NS
    wproj, eproj = _make_sc_proj(v)(
        word_table.reshape(-1), entity_table.reshape(-1), erw_b, elw_b)
    wsum, esum = _make_sc_gather(v, rows, cw)(wproj, eproj, widx_t, eidx_t)

    return _epilogue(wsum, esum, er_b, el_b, ec_W, cluster_W, cluster_b)
